# baseline (speedup 1.0000x reference)
import functools

import jax
import jax.numpy as jnp
from jax import lax
from jax.experimental import pallas as pl
from jax.experimental.pallas import tpu as pltpu
from jax.experimental.pallas import tpu_sc as plsc

_L = 16
_NC = 2
_NS = 16
_NW = _NC * _NS

_CHUNK = 128
_NBUF = 8


@functools.lru_cache(maxsize=None)
def _make_gather(R, D, C, V, B):
    assert R % (_NW * _CHUNK) == 0
    assert B & (B - 1) == 0, "batch must be a power of two"
    _LOGB = B.bit_length() - 1
    rpw = R // _NW
    nch = rpw // _CHUNK
    assert nch % _NBUF == 0
    ngroups = nch // _NBUF

    mesh = plsc.VectorSubcoreMesh(core_axis_name="c", subcore_axis_name="s")

    @functools.partial(
        pl.kernel,
        mesh=mesh,
        out_type=jax.ShapeDtypeStruct((R, D), jnp.float32),
        scratch_types=(
            [pltpu.VMEM((nch, _CHUNK), jnp.float32),
             pltpu.VMEM((nch, _CHUNK), jnp.int32),
             pltpu.VMEM((nch, _CHUNK), jnp.int32)]
            + [pltpu.VMEM((_CHUNK, D), jnp.float32) for _ in range(_NBUF)]
            + [pltpu.SemaphoreType.DMA for _ in range(2 * _NBUF)]
        ),
        compiler_params=pltpu.CompilerParams(use_tc_tiling_on_sc=False),
    )
    def gather_kernel(tab_hbm, idx_hbm, out_hbm, idxf, idx2d, orow2d, *rest):
        rows = rest[:_NBUF]
        gsem = rest[_NBUF:2 * _NBUF]
        ssem = rest[2 * _NBUF:]

        wid = lax.axis_index("s") * _NC + lax.axis_index("c")
        chunk0 = wid * nch
        row0 = wid * rpw

        pltpu.sync_copy(idx_hbm.at[pl.ds(chunk0, nch)], idxf)

        lanes = lax.iota(jnp.int32, _L)

        def tbody(j, carry):
            r_base = row0 + j * _CHUNK
            for i in range(_CHUNK // _L):
                v = idxf[j, pl.ds(i * _L, _L)].astype(jnp.int32)
                v = jnp.minimum(jnp.maximum(v, 0), V - 1)
                r = r_base + i * _L + lanes
                c = lax.shift_right_logical(r, _LOGB)
                idx2d[j, pl.ds(i * _L, _L)] = v + c * V
                orow2d[j, pl.ds(i * _L, _L)] = (r - c * B) * C + c
            return carry

        lax.fori_loop(0, nch, tbody, 0)

        def g_start(j, b):
            pltpu.make_async_copy(
                tab_hbm.at[idx2d.at[j]], rows[b], gsem[b]).start()

        def g_wait(j, b):
            pltpu.make_async_copy(
                tab_hbm.at[idx2d.at[j]], rows[b], gsem[b]).wait()

        def s_start(j, b):
            pltpu.make_async_copy(
                rows[b], out_hbm.at[orow2d.at[j]], ssem[b]).start()

        def s_wait(j, b):
            pltpu.make_async_copy(
                rows[b], out_hbm.at[orow2d.at[j]], ssem[b]).wait()

        for b in range(_NBUF):
            g_start(b, b)

        def group(gi, carry):
            g = gi * _NBUF
            for b in range(_NBUF):
                j = g + b
                g_wait(j, b)
                s_start(j, b)
            for b in range(_NBUF):
                j = g + b
                s_wait(j, b)
                g_start(j + _NBUF, b)
            return carry

        lax.fori_loop(0, ngroups - 1, group, 0)

        g = (ngroups - 1) * _NBUF
        for b in range(_NBUF):
            j = g + b
            g_wait(j, b)
            s_start(j, b)
        for b in range(_NBUF):
            s_wait(g + b, b)

    return gather_kernel


def kernel(cat_tensor, tables):
    if cat_tensor.ndim == 1:
        cat_tensor = cat_tensor[None, :]
    B, C = cat_tensor.shape
    _, V, D = tables.shape
    R = B * C
    tab_flat = tables.reshape(C * V, D)
    idx = lax.optimization_barrier(
        cat_tensor.T.astype(jnp.float32).reshape(R // _CHUNK, _CHUNK))
    out = _make_gather(R, D, C, V, B)(tab_flat, idx)
    return out.reshape(B, C * D)


# speedup vs baseline: 1.0029x; 1.0029x over previous
import functools

import jax
import jax.numpy as jnp
from jax import lax
from jax.experimental import pallas as pl
from jax.experimental.pallas import tpu as pltpu
from jax.experimental.pallas import tpu_sc as plsc

_NC = 2
_NS = 16
_NW = _NC * _NS

_CHUNK = 128
_NBUF = 8


@functools.lru_cache(maxsize=None)
def _make_gather(R, D):
    assert R % (_NW * _CHUNK) == 0
    rpw = R // _NW
    nch = rpw // _CHUNK
    assert nch % _NBUF == 0
    ngroups = nch // _NBUF

    mesh = plsc.VectorSubcoreMesh(core_axis_name="c", subcore_axis_name="s")

    @functools.partial(
        pl.kernel,
        mesh=mesh,
        out_type=jax.ShapeDtypeStruct((R, D), jnp.float32),
        scratch_types=(
            [pltpu.VMEM((nch, _CHUNK), jnp.int32)]
            + [pltpu.VMEM((_CHUNK, D), jnp.float32) for _ in range(_NBUF)]
            + [pltpu.SemaphoreType.DMA for _ in range(2 * _NBUF)]
        ),
        compiler_params=pltpu.CompilerParams(use_tc_tiling_on_sc=False),
    )
    def gather_kernel(tab_hbm, idx_hbm, out_hbm, idx2d, *rest):
        rows = rest[:_NBUF]
        gsem = rest[_NBUF:2 * _NBUF]
        ssem = rest[2 * _NBUF:]

        wid = lax.axis_index("s") * _NC + lax.axis_index("c")
        chunk0 = wid * nch
        row0 = wid * rpw

        pltpu.sync_copy(idx_hbm.at[pl.ds(chunk0, nch)], idx2d)

        def g_start(j, b):
            pltpu.make_async_copy(
                tab_hbm.at[idx2d.at[j]], rows[b], gsem[b]).start()

        def g_wait(j, b):
            pltpu.make_async_copy(
                tab_hbm.at[idx2d.at[j]], rows[b], gsem[b]).wait()

        def s_start(j, b):
            pltpu.make_async_copy(
                rows[b], out_hbm.at[pl.ds(row0 + j * _CHUNK, _CHUNK)],
                ssem[b]).start()

        def s_wait(j, b):
            pltpu.make_async_copy(
                rows[b], out_hbm.at[pl.ds(row0 + j * _CHUNK, _CHUNK)],
                ssem[b]).wait()

        for b in range(_NBUF):
            g_start(b, b)

        def group(gi, carry):
            g = gi * _NBUF
            for b in range(_NBUF):
                j = g + b
                g_wait(j, b)
                s_start(j, b)
            for b in range(_NBUF):
                j = g + b
                s_wait(j, b)
                g_start(j + _NBUF, b)
            return carry

        lax.fori_loop(0, ngroups - 1, group, 0)

        g = (ngroups - 1) * _NBUF
        for b in range(_NBUF):
            j = g + b
            g_wait(j, b)
            s_start(j, b)
        for b in range(_NBUF):
            s_wait(g + b, b)

    return gather_kernel


def kernel(cat_tensor, tables):
    if cat_tensor.ndim == 1:
        cat_tensor = cat_tensor[None, :]
    B, C = cat_tensor.shape
    _, V, D = tables.shape
    R = B * C
    tab_flat = tables.reshape(C * V, D)
    base = jnp.arange(C, dtype=cat_tensor.dtype) * V
    flat_idx = jnp.clip(cat_tensor, 0, V - 1) + base[None, :]
    idx = lax.optimization_barrier(
        flat_idx.astype(jnp.int32).reshape(R // _CHUNK, _CHUNK))
    out = _make_gather(R, D)(tab_flat, idx)
    return out.reshape(B, C * D)
